# baseline (speedup 1.0000x reference)
import jax
import jax.numpy as jnp
from jax import lax
from jax.experimental import pallas as pl
from jax.experimental.pallas import tpu as pltpu
from jax.experimental.pallas import tpu_sc as plsc

ROWS = 128
COLS = 32768
K = 64
LANES = 16
NV = COLS // LANES
UNROLL = 8
NWORKERS = 32
ROWS_PER = ROWS // NWORKERS
LANE_CAP = 64
CAP = LANES * LANE_CAP
NCV = CAP // LANES


def _keys(xb):
    sign = jnp.uint32(0x80000000)
    b = lax.bitcast_convert_type(xb, jnp.uint32)
    neg = b >= sign
    return jnp.where(neg, ~b, b | sign)


def _unkey_f(kv):
    sign = jnp.uint32(0x80000000)
    bits = jnp.where(kv >= sign, kv ^ sign, ~kv)
    return plsc.bitcast(bits, jnp.float32)


def _process_row(xb, cv, ci, ov, wait_out=None):
    def a_body(i, carry):
        a0, a1, a2, a3 = carry
        for g in range(UNROLL // 4):
            base = (i * UNROLL + g * 4) * LANES
            v0 = xb[pl.ds(base, LANES)]
            v1 = xb[pl.ds(base + LANES, LANES)]
            v2 = xb[pl.ds(base + 2 * LANES, LANES)]
            v3 = xb[pl.ds(base + 3 * LANES, LANES)]
            t = jnp.maximum(jnp.maximum(v0, v1), jnp.maximum(v2, v3))
            m = jnp.maximum(a0, t); t = jnp.minimum(a0, t); a0 = m
            m = jnp.maximum(a1, t); t = jnp.minimum(a1, t); a1 = m
            m = jnp.maximum(a2, t); t = jnp.minimum(a2, t); a2 = m
            a3 = jnp.maximum(a3, t)
        return (a0, a1, a2, a3)

    ninf = [jnp.full((LANES,), -jnp.inf, jnp.float32) for _ in range(4)]
    a0, _, _, a3 = lax.fori_loop(0, NV // UNROLL, a_body, tuple(ninf))
    t2s = jnp.full((LANES,), jnp.min(a3), dtype=jnp.float32)
    mxs = jnp.full((LANES,), jnp.max(a0), dtype=jnp.float32)

    def fill_body(i, _):
        for u in range(UNROLL):
            cv[pl.ds((i * UNROLL + u) * LANES, LANES)] = (
                jnp.full((LANES,), -jnp.inf, jnp.float32))
        return 0

    lax.fori_loop(0, NCV // UNROLL, fill_body, 0)

    lane = lax.iota(jnp.int32, LANES)
    capm1 = jnp.full((LANES,), LANE_CAP - 1, jnp.int32)

    def b_body(i, off):
        for u in range(UNROLL):
            vi = i * UNROLL + u
            sl = pl.ds(vi * LANES, LANES)
            x = xb[sl]
            msk = x >= t2s
            idx = jnp.minimum(off, capm1) * LANES + lane
            plsc.store_scatter(cv, [idx], x, mask=msk)
            eidx = jnp.full((LANES,), vi * LANES, jnp.int32) + lane
            plsc.store_scatter(ci, [idx], eidx, mask=msk)
            off = off + msk.astype(jnp.int32)
        return off

    lax.fori_loop(0, NV // UNROLL, b_body, jnp.zeros((LANES,), jnp.int32))

    kones = jnp.full((LANES,), 0xFFFFFFFF, dtype=jnp.uint32)
    kmax = _keys(mxs)
    hi0 = jnp.where(kmax == kones, kones, kmax + jnp.uint32(1))
    lo0 = _keys(t2s)
    ksp = jnp.full((LANES,), K, jnp.int32)

    def count_ge(thrf):
        def cb(i, acc):
            for u in range(4):
                sl = pl.ds((i * 4 + u) * LANES, LANES)
                acc = acc + plsc.all_reduce_population_count(cv[sl] >= thrf)
            return acc
        return lax.fori_loop(0, NCV // 4, cb, jnp.zeros((LANES,), jnp.int32))

    def s_body(j, carry):
        lo, hi = carry
        mid = lo + ((hi - lo) >> jnp.uint32(1))
        cnt = count_ge(_unkey_f(mid))
        ge = cnt >= ksp
        return (jnp.where(ge, mid, lo), jnp.where(ge, hi, mid))

    lo, _ = lax.fori_loop(0, 32, s_body, (lo0, hi0))
    thrf = _unkey_f(lo)

    def count_gt(thr):
        def cb(i, acc):
            for u in range(4):
                sl = pl.ds((i * 4 + u) * LANES, LANES)
                acc = acc + plsc.all_reduce_population_count(cv[sl] > thr)
            return acc
        return lax.fori_loop(0, NCV // 4, cb, jnp.zeros((LANES,), jnp.int32))

    needed = ksp - count_gt(thrf)

    def count_eq_le(im):
        def cb(i, acc):
            for u in range(4):
                sl = pl.ds((i * 4 + u) * LANES, LANES)
                hit = jnp.logical_and(cv[sl] == thrf, ci[sl] <= im)
                acc = acc + plsc.all_reduce_population_count(hit)
            return acc
        return lax.fori_loop(0, NCV // 4, cb, jnp.zeros((LANES,), jnp.int32))

    def t_body(j, carry):
        lo2, hi2 = carry
        mid2 = (lo2 + hi2) >> jnp.int32(1)
        ge2 = count_eq_le(mid2) >= needed
        return (jnp.where(ge2, lo2, mid2 + jnp.int32(1)),
                jnp.where(ge2, mid2, hi2))

    _, istar = lax.fori_loop(
        0, 15, t_body,
        (jnp.zeros((LANES,), jnp.int32),
         jnp.full((LANES,), COLS - 1, jnp.int32)))

    if wait_out is not None:
        wait_out()

    def mask_body(i, _):
        for u in range(UNROLL):
            vi = i * UNROLL + u
            sl = pl.ds(vi * LANES, LANES)
            x = xb[sl]
            eidx = jnp.full((LANES,), vi * LANES, jnp.int32) + lane
            keep = jnp.logical_or(
                x > thrf,
                jnp.logical_and(x == thrf, eidx <= istar))
            ov[sl] = jnp.where(keep, x, jnp.float32(0.0))
        return 0

    lax.fori_loop(0, NV // UNROLL, mask_body, 0)


def _body(x_hbm, out_hbm, x0, x1, ov, cv, ci, sin0, sin1, sout):
    wid = lax.axis_index("s") * 2 + lax.axis_index("c")
    row0 = wid * ROWS_PER
    bufs = (x0, x1)
    sems = (sin0, sin1)

    in_handles = [None, None]
    in_handles[0] = pltpu.async_copy(x_hbm.at[row0], x0, sin0)
    out_handle = None
    for r in range(ROWS_PER):
        xb = bufs[r % 2]
        if r + 1 < ROWS_PER:
            in_handles[(r + 1) % 2] = pltpu.async_copy(
                x_hbm.at[row0 + r + 1], bufs[(r + 1) % 2],
                sems[(r + 1) % 2])
        in_handles[r % 2].wait()
        _process_row(xb, cv, ci, ov,
                     wait_out=out_handle.wait if out_handle else None)
        out_handle = pltpu.async_copy(ov, out_hbm.at[row0 + r], sout)
    out_handle.wait()


def kernel(x):
    mesh = plsc.VectorSubcoreMesh(core_axis_name="c", subcore_axis_name="s")
    f = pl.kernel(
        _body,
        mesh=mesh,
        out_type=jax.ShapeDtypeStruct((ROWS, COLS), jnp.float32),
        scratch_types=[
            pltpu.VMEM((COLS,), jnp.float32),
            pltpu.VMEM((COLS,), jnp.float32),
            pltpu.VMEM((COLS,), jnp.float32),
            pltpu.VMEM((CAP,), jnp.float32),
            pltpu.VMEM((CAP,), jnp.int32),
            pltpu.SemaphoreType.DMA,
            pltpu.SemaphoreType.DMA,
            pltpu.SemaphoreType.DMA,
        ],
        compiler_params=pltpu.CompilerParams(needs_layout_passes=False),
    )
    return f(x)
